# Initial kernel scaffold; baseline (speedup 1.0000x reference)
import functools

import jax
import jax.numpy as jnp
from jax import lax
from jax.experimental import pallas as pl
from jax.experimental.pallas import tpu as pltpu
from jax.experimental.pallas import tpu_sc as plsc

N = 10000
E = 320000
D_IN = 128
D_HID = 128
D_OUT = 64

NC = 2
NS = 16
NW = NC * NS
K = 128
C = 80
E_PAD = NW * C * K
N_PAD = 10240
RPT = N_PAD // NS
ZR = 64

R_BLK = 1024
G = N_PAD // R_BLK


def _mesh():
    return plsc.VectorSubcoreMesh(core_axis_name="c", subcore_axis_name="s")


def _sc_degree(dst_p):
    @functools.partial(
        pl.kernel,
        out_type=jax.ShapeDtypeStruct((NC, N_PAD, 16), jnp.float32),
        mesh=_mesh(),
        scratch_types=[
            pltpu.VMEM((C, K), jnp.int32),
            pltpu.VMEM((K, 16), jnp.float32),
            pltpu.VMEM((RPT, 16), jnp.float32),
            pltpu.VMEM_SHARED((N_PAD, 16), jnp.float32),
        ],
    )
    def k(dst_hbm, out_hbm, idx_v, e0_v, buf_v, acc_sh):
        cid = lax.axis_index("c")
        sid = lax.axis_index("s")
        wid = cid * NS + sid
        pltpu.sync_copy(dst_hbm.at[wid], idx_v)

        e0 = (lax.iota(jnp.int32, 16) == 0).astype(jnp.float32)
        z16 = jnp.zeros((16,), jnp.float32)

        @pl.loop(0, K)
        def _(r):
            e0_v[r, :] = e0

        @pl.loop(0, RPT)
        def _(r):
            buf_v[r, :] = z16

        pltpu.sync_copy(buf_v, acc_sh.at[pl.ds(sid * RPT, RPT)])
        plsc.subcore_barrier()

        @pl.loop(0, C)
        def _(j):
            pltpu.sync_copy(e0_v, acc_sh.at[idx_v.at[j]], add=True)

        plsc.subcore_barrier()
        pltpu.sync_copy(acc_sh.at[pl.ds(sid * RPT, RPT)], buf_v)
        pltpu.sync_copy(buf_v, out_hbm.at[cid, pl.ds(sid * RPT, RPT)])

    return k(dst_p)


def _sc_aggregate(y_p, src_p, dst_p, d):
    @functools.partial(
        pl.kernel,
        out_type=jax.ShapeDtypeStruct((NC, N_PAD, d), jnp.float32),
        mesh=_mesh(),
        scratch_types=[
            pltpu.VMEM((C, K), jnp.int32),
            pltpu.VMEM((C, K), jnp.int32),
            pltpu.VMEM((K, d), jnp.float32),
            pltpu.VMEM((K, d), jnp.float32),
            pltpu.VMEM((ZR, d), jnp.float32),
            pltpu.VMEM_SHARED((N_PAD, d), jnp.float32),
            pltpu.SemaphoreType.DMA,
            pltpu.SemaphoreType.DMA,
        ],
    )
    def k(y_hbm, src_hbm, dst_hbm, out_hbm,
          src_v, dst_v, ra, rb, zb, acc_sh, sga, sgb):
        cid = lax.axis_index("c")
        sid = lax.axis_index("s")
        wid = cid * NS + sid
        pltpu.sync_copy(src_hbm.at[wid], src_v)
        pltpu.sync_copy(dst_hbm.at[wid], dst_v)

        z16 = jnp.zeros((16,), jnp.float32)

        @pl.loop(0, ZR)
        def _(r):
            @pl.loop(0, d, step=16)
            def _(cc):
                zb[r, pl.ds(cc, 16)] = z16

        @pl.loop(0, RPT, step=ZR)
        def _(r):
            pltpu.sync_copy(zb, acc_sh.at[pl.ds(sid * RPT + r, ZR)])

        plsc.subcore_barrier()

        pltpu.async_copy(y_hbm.at[src_v.at[0]], ra, sga)
        pltpu.async_copy(y_hbm.at[src_v.at[1]], rb, sgb)

        @pl.loop(0, C, step=2)
        def _(j):
            pltpu.make_async_copy(y_hbm.at[src_v.at[j]], ra, sga).wait()
            pltpu.sync_copy(ra, acc_sh.at[dst_v.at[j]], add=True)

            @pl.when(j + 2 < C)
            def _():
                pltpu.async_copy(y_hbm.at[src_v.at[j + 2]], ra, sga)

            pltpu.make_async_copy(y_hbm.at[src_v.at[j + 1]], rb, sgb).wait()
            pltpu.sync_copy(rb, acc_sh.at[dst_v.at[j + 1]], add=True)

            @pl.when(j + 3 < C)
            def _():
                pltpu.async_copy(y_hbm.at[src_v.at[j + 3]], rb, sgb)

        plsc.subcore_barrier()

        @pl.loop(0, RPT, step=ZR)
        def _(r):
            pltpu.sync_copy(acc_sh.at[pl.ds(sid * RPT + r, ZR)], zb)
            pltpu.sync_copy(zb, out_hbm.at[cid, pl.ds(sid * RPT + r, ZR)])

    return k(y_p, src_p, dst_p)


def _tc_pre(deg_a, deg_b, x_p, W1):
    def body(dega_r, degb_r, x_r, w_r, y_r, dis_r):
        deg = jnp.sum(dega_r[...], axis=1) + jnp.sum(degb_r[...], axis=1) + 1.0
        dis = lax.rsqrt(deg)
        h = jnp.dot(x_r[...], w_r[...], preferred_element_type=jnp.float32)
        y_r[...] = h * dis[:, None]
        dis_r[...] = dis

    return pl.pallas_call(
        body,
        grid=(G,),
        in_specs=[
            pl.BlockSpec((R_BLK, 16), lambda i: (i, 0)),
            pl.BlockSpec((R_BLK, 16), lambda i: (i, 0)),
            pl.BlockSpec((R_BLK, D_IN), lambda i: (i, 0)),
            pl.BlockSpec((D_IN, D_HID), lambda i: (0, 0)),
        ],
        out_specs=[
            pl.BlockSpec((R_BLK, D_HID), lambda i: (i, 0)),
            pl.BlockSpec((R_BLK,), lambda i: (i,)),
        ],
        out_shape=[
            jax.ShapeDtypeStruct((N_PAD, D_HID), jnp.float32),
            jax.ShapeDtypeStruct((N_PAD,), jnp.float32),
        ],
    )(deg_a, deg_b, x_p, W1)


def _tc_mid(acc_a, acc_b, y1, dis, b1, W2):
    def body(aa_r, ab_r, y1_r, dis_r, b1_r, w2_r, hid_r, y2_r):
        s = (aa_r[...] + ab_r[...] + y1_r[...]) * dis_r[...][:, None]
        h = jnp.maximum(s + b1_r[...][None, :], 0.0)
        hid_r[...] = h
        m = jnp.dot(h, w2_r[...], preferred_element_type=jnp.float32)
        y2_r[...] = m * dis_r[...][:, None]

    return pl.pallas_call(
        body,
        grid=(G,),
        in_specs=[
            pl.BlockSpec((R_BLK, D_HID), lambda i: (i, 0)),
            pl.BlockSpec((R_BLK, D_HID), lambda i: (i, 0)),
            pl.BlockSpec((R_BLK, D_HID), lambda i: (i, 0)),
            pl.BlockSpec((R_BLK,), lambda i: (i,)),
            pl.BlockSpec((D_HID,), lambda i: (0,)),
            pl.BlockSpec((D_HID, D_OUT), lambda i: (0, 0)),
        ],
        out_specs=[
            pl.BlockSpec((R_BLK, D_HID), lambda i: (i, 0)),
            pl.BlockSpec((R_BLK, D_OUT), lambda i: (i, 0)),
        ],
        out_shape=[
            jax.ShapeDtypeStruct((N_PAD, D_HID), jnp.float32),
            jax.ShapeDtypeStruct((N_PAD, D_OUT), jnp.float32),
        ],
    )(acc_a, acc_b, y1, dis, b1, W2)


def _tc_post(acc_a, acc_b, y2, dis, b2):
    def body(aa_r, ab_r, y2_r, dis_r, b2_r, o_r):
        s = (aa_r[...] + ab_r[...] + y2_r[...]) * dis_r[...][:, None]
        o_r[...] = s + b2_r[...][None, :]

    return pl.pallas_call(
        body,
        grid=(G,),
        in_specs=[
            pl.BlockSpec((R_BLK, D_OUT), lambda i: (i, 0)),
            pl.BlockSpec((R_BLK, D_OUT), lambda i: (i, 0)),
            pl.BlockSpec((R_BLK, D_OUT), lambda i: (i, 0)),
            pl.BlockSpec((R_BLK,), lambda i: (i,)),
            pl.BlockSpec((D_OUT,), lambda i: (0,)),
        ],
        out_specs=pl.BlockSpec((R_BLK, D_OUT), lambda i: (i, 0)),
        out_shape=jax.ShapeDtypeStruct((N_PAD, D_OUT), jnp.float32),
    )(acc_a, acc_b, y2, dis, b2)


def kernel(x, edge_index, W1, b1, W2, b2):
    src = edge_index[0].astype(jnp.int32)
    dst = edge_index[1].astype(jnp.int32)
    pad = jnp.full((E_PAD - E,), N, dtype=jnp.int32)
    src_p = jnp.concatenate([src, pad]).reshape(NW, C, K)
    dst_p = jnp.concatenate([dst, pad]).reshape(NW, C, K)
    x_p = jnp.zeros((N_PAD, D_IN), jnp.float32).at[:N].set(x)

    deg16 = _sc_degree(dst_p)

    y1, dis = _tc_pre(deg16[0], deg16[1], x_p, W1)

    acc1 = _sc_aggregate(y1, src_p, dst_p, D_HID)

    hidden_p, y2 = _tc_mid(acc1[0], acc1[1], y1, dis, b1, W2)

    acc2 = _sc_aggregate(y2, src_p, dst_p, D_OUT)

    out_p = _tc_post(acc2[0], acc2[1], y2, dis, b2)

    return (out_p[:N], hidden_p[:N])


# speedup vs baseline: 5.3359x; 5.3359x over previous
import functools

import jax
import jax.numpy as jnp
from jax import lax
from jax.experimental import pallas as pl
from jax.experimental.pallas import tpu as pltpu
from jax.experimental.pallas import tpu_sc as plsc

N = 10000
E = 320000
D_IN = 128
D_HID = 128
D_OUT = 64

NC = 2
NS = 16
NW = NC * NS
K = 128
C = 80
E_PAD = NW * C * K
C2 = E_PAD // (NS * K)
N_PAD = 10240
ZR = 64
DA = 128
SHARD = 5056
BLK_SH = SHARD // ZR
ACC_R = SHARD + 16

R_BLK = 1024
G = N_PAD // R_BLK


def _mesh():
    return plsc.VectorSubcoreMesh(core_axis_name="c", subcore_axis_name="s")


_SC_PARAMS = pltpu.CompilerParams(needs_layout_passes=False)


def _sc_degree(dst_p, e0):
    @functools.partial(
        pl.kernel,
        out_type=jax.ShapeDtypeStruct((N_PAD, DA), jnp.float32),
        mesh=_mesh(),
        compiler_params=_SC_PARAMS,
        scratch_types=[
            pltpu.VMEM((C2, K), jnp.int32),
            pltpu.VMEM((K,), jnp.int32),
            pltpu.VMEM((K, DA), jnp.float32),
            pltpu.VMEM((ZR, DA), jnp.float32),
            pltpu.VMEM((ZR, DA), jnp.float32),
            pltpu.VMEM_SHARED((ACC_R, DA), jnp.float32),
        ],
    )
    def k(dst_hbm, e0_hbm, out_hbm, idx_v, jbuf, e0_v, zb, ob, acc_sh):
        cid = lax.axis_index("c")
        sid = lax.axis_index("s")
        base = cid * SHARD
        pltpu.sync_copy(dst_hbm.at[cid, sid], idx_v)
        pltpu.sync_copy(e0_hbm, e0_v)

        z16 = jnp.zeros((16,), jnp.float32)

        @pl.loop(0, ZR)
        def _(r):
            @pl.loop(0, DA, step=16)
            def _(cc):
                zb[r, pl.ds(cc, 16)] = z16

        @pl.loop(0, 5)
        def _(r):
            b = r * NS + sid

            @pl.when(b < BLK_SH)
            def _():
                pltpu.sync_copy(zb, acc_sh.at[pl.ds(b * ZR, ZR)])

        plsc.subcore_barrier()

        @pl.loop(0, C2)
        def _(j):
            @pl.loop(0, K, step=16)
            def _(q):
                jbuf[pl.ds(q, 16)] = idx_v[j, pl.ds(q, 16)]

            pltpu.sync_copy(e0_v, acc_sh.at[jbuf], add=True)

        plsc.subcore_barrier()

        @pl.loop(0, 5)
        def _(r):
            b = r * NS + sid

            @pl.when(b < BLK_SH)
            def _():
                pltpu.sync_copy(acc_sh.at[pl.ds(b * ZR, ZR)], ob)
                pltpu.sync_copy(ob, out_hbm.at[pl.ds(base + b * ZR, ZR)])

    return k(dst_p, e0)


def _sc_aggregate(y, src_p, dst_p):
    @functools.partial(
        pl.kernel,
        out_type=jax.ShapeDtypeStruct((N_PAD, DA), jnp.float32),
        mesh=_mesh(),
        compiler_params=_SC_PARAMS,
        scratch_types=[
            pltpu.VMEM((C2, K), jnp.int32),
            pltpu.VMEM((C2, K), jnp.int32),
            pltpu.VMEM((K,), jnp.int32),
            pltpu.VMEM((K, DA), jnp.float32),
            pltpu.VMEM((K, DA), jnp.float32),
            pltpu.VMEM((ZR, DA), jnp.float32),
            pltpu.VMEM((ZR, DA), jnp.float32),
            pltpu.VMEM_SHARED((ACC_R, DA), jnp.float32),
            pltpu.SemaphoreType.DMA,
            pltpu.SemaphoreType.DMA,
        ],
    )
    def k(y_hbm, src_hbm, dst_hbm, out_hbm, src_v, dst_v, jbuf,
          ra, rb, zb, ob, acc_sh, sga, sgb):
        cid = lax.axis_index("c")
        sid = lax.axis_index("s")
        base = cid * SHARD
        pltpu.sync_copy(src_hbm.at[sid], src_v)
        pltpu.sync_copy(dst_hbm.at[cid, sid], dst_v)

        z16 = jnp.zeros((16,), jnp.float32)

        @pl.loop(0, ZR)
        def _(r):
            @pl.loop(0, DA, step=16)
            def _(cc):
                zb[r, pl.ds(cc, 16)] = z16

        @pl.loop(0, 5)
        def _(r):
            b = r * NS + sid

            @pl.when(b < BLK_SH)
            def _():
                pltpu.sync_copy(zb, acc_sh.at[pl.ds(b * ZR, ZR)])

        plsc.subcore_barrier()

        pltpu.async_copy(y_hbm.at[src_v.at[0]], ra, sga)
        pltpu.async_copy(y_hbm.at[src_v.at[1]], rb, sgb)

        @pl.loop(0, C2, step=2)
        def _(j):
            pltpu.make_async_copy(y_hbm.at[pl.ds(0, K)], ra, sga).wait()

            @pl.loop(0, K, step=16)
            def _(q):
                jbuf[pl.ds(q, 16)] = dst_v[j, pl.ds(q, 16)]

            pltpu.sync_copy(ra, acc_sh.at[jbuf], add=True)

            @pl.when(j + 2 < C2)
            def _():
                pltpu.async_copy(y_hbm.at[src_v.at[j + 2]], ra, sga)

            pltpu.make_async_copy(y_hbm.at[pl.ds(0, K)], rb, sgb).wait()

            @pl.loop(0, K, step=16)
            def _(q):
                jbuf[pl.ds(q, 16)] = dst_v[j + 1, pl.ds(q, 16)]

            pltpu.sync_copy(rb, acc_sh.at[jbuf], add=True)

            @pl.when(j + 3 < C2)
            def _():
                pltpu.async_copy(y_hbm.at[src_v.at[j + 3]], rb, sgb)

        plsc.subcore_barrier()

        @pl.loop(0, 5)
        def _(r):
            b = r * NS + sid

            @pl.when(b < BLK_SH)
            def _():
                pltpu.sync_copy(acc_sh.at[pl.ds(b * ZR, ZR)], ob)
                pltpu.sync_copy(ob, out_hbm.at[pl.ds(base + b * ZR, ZR)])

    return k(y, src_p, dst_p)


def _tc_pre(deg, x_p, W1):
    def body(deg_r, x_r, w_r, y_r, dis_r):
        dis = lax.rsqrt(deg_r[...][:, 0] + 1.0)
        h = jnp.dot(x_r[...], w_r[...], preferred_element_type=jnp.float32)
        y_r[...] = h * dis[:, None]
        dis_r[...] = dis

    return pl.pallas_call(
        body,
        grid=(G,),
        in_specs=[
            pl.BlockSpec((R_BLK, DA), lambda i: (i, 0)),
            pl.BlockSpec((R_BLK, D_IN), lambda i: (i, 0)),
            pl.BlockSpec((D_IN, D_HID), lambda i: (0, 0)),
        ],
        out_specs=[
            pl.BlockSpec((R_BLK, D_HID), lambda i: (i, 0)),
            pl.BlockSpec((R_BLK,), lambda i: (i,)),
        ],
        out_shape=[
            jax.ShapeDtypeStruct((N_PAD, D_HID), jnp.float32),
            jax.ShapeDtypeStruct((N_PAD,), jnp.float32),
        ],
    )(deg, x_p, W1)


def _tc_mid(acc1, y1, dis, b1, W2):
    def body(a_r, y1_r, dis_r, b1_r, w2_r, hid_r, y2_r):
        ds = dis_r[...][:, None]
        s = (a_r[...] + y1_r[...]) * ds
        hfull = jnp.maximum(s + b1_r[...][None, :], 0.0)
        hid_r[...] = hfull
        m = jnp.dot(hfull, w2_r[...], preferred_element_type=jnp.float32)
        y2_r[...] = jnp.pad(m * ds, ((0, 0), (0, DA - D_OUT)))

    return pl.pallas_call(
        body,
        grid=(G,),
        in_specs=[
            pl.BlockSpec((R_BLK, DA), lambda i: (i, 0)),
            pl.BlockSpec((R_BLK, D_HID), lambda i: (i, 0)),
            pl.BlockSpec((R_BLK,), lambda i: (i,)),
            pl.BlockSpec((D_HID,), lambda i: (0,)),
            pl.BlockSpec((D_HID, D_OUT), lambda i: (0, 0)),
        ],
        out_specs=[
            pl.BlockSpec((R_BLK, D_HID), lambda i: (i, 0)),
            pl.BlockSpec((R_BLK, DA), lambda i: (i, 0)),
        ],
        out_shape=[
            jax.ShapeDtypeStruct((N_PAD, D_HID), jnp.float32),
            jax.ShapeDtypeStruct((N_PAD, DA), jnp.float32),
        ],
    )(acc1, y1, dis, b1, W2)


def _tc_post(acc2, y2, dis, b2):
    def body(a_r, y2_r, dis_r, b2_r, o_r):
        s = a_r[...] + y2_r[...]
        o_r[...] = s[:, :D_OUT] * dis_r[...][:, None] + b2_r[...][None, :]

    return pl.pallas_call(
        body,
        grid=(G,),
        in_specs=[
            pl.BlockSpec((R_BLK, DA), lambda i: (i, 0)),
            pl.BlockSpec((R_BLK, DA), lambda i: (i, 0)),
            pl.BlockSpec((R_BLK,), lambda i: (i,)),
            pl.BlockSpec((D_OUT,), lambda i: (0,)),
        ],
        out_specs=pl.BlockSpec((R_BLK, D_OUT), lambda i: (i, 0)),
        out_shape=jax.ShapeDtypeStruct((N_PAD, D_OUT), jnp.float32),
    )(acc2, y2, dis, b2)


def kernel(x, edge_index, W1, b1, W2, b2):
    src = edge_index[0].astype(jnp.int32)
    dst = edge_index[1].astype(jnp.int32)
    pad = jnp.full((E_PAD - E,), N, dtype=jnp.int32)
    src_f = jnp.concatenate([src, pad])
    dst_f = jnp.concatenate([dst, pad])
    src_a = src_f.reshape(NS, C2, K)
    rel = dst_f[None, :] - (jnp.arange(NC, dtype=jnp.int32) * SHARD)[:, None]
    dst_a = jnp.where(
        (rel >= 0) & (rel < SHARD), rel, SHARD
    ).astype(jnp.int32).reshape(NC, NS, C2, K)
    e0 = jnp.zeros((K, DA), jnp.float32).at[:, 0].set(1.0)
    x_p = jnp.zeros((N_PAD, D_IN), jnp.float32).at[:N].set(x)

    deg = _sc_degree(dst_a, e0)

    y1, dis = _tc_pre(deg, x_p, W1)

    acc1 = _sc_aggregate(y1, src_a, dst_a)

    hidden_p, y2 = _tc_mid(acc1, y1, dis, b1, W2)

    acc2 = _sc_aggregate(y2, src_a, dst_a)

    out_p = _tc_post(acc2, y2, dis, b2)

    return (out_p[:N], hidden_p[:N])
